# baseline (speedup 1.0000x reference)
import jax
from jax.experimental import pallas as pl
from jax.experimental.pallas import tpu as pltpu

_BLOCK_ROWS = 16384


def _copy_block(x_ref, o_ref):
    o_ref[...] = x_ref[...]


def kernel(data, partitions):
    del partitions
    n, d = data.shape
    return pl.pallas_call(
        _copy_block,
        grid=(n // _BLOCK_ROWS,),
        in_specs=[pl.BlockSpec((_BLOCK_ROWS, d), lambda i: (i, 0))],
        out_specs=pl.BlockSpec((_BLOCK_ROWS, d), lambda i: (i, 0)),
        out_shape=jax.ShapeDtypeStruct((n, d), data.dtype),
        compiler_params=pltpu.CompilerParams(
            dimension_semantics=("parallel",),
            vmem_limit_bytes=100 * 1024 * 1024,
        ),
    )(data)


# speedup vs baseline: 1.0006x; 1.0006x over previous
import jax
import jax.numpy as jnp
from jax.experimental import pallas as pl
from jax.experimental.pallas import tpu as pltpu

_CHUNK_ROWS = 16384
_NSLOTS = 4


def _make_copy_kernel(nchunks):
    lookahead = _NSLOTS - 1

    def _copy(x_hbm, o_hbm, buf, in_sem, out_sem):
        def in_copy(i, slot):
            return pltpu.make_async_copy(
                x_hbm.at[pl.ds(i * _CHUNK_ROWS, _CHUNK_ROWS)],
                buf.at[slot], in_sem.at[slot])

        def out_copy(i, slot):
            return pltpu.make_async_copy(
                buf.at[slot],
                o_hbm.at[pl.ds(i * _CHUNK_ROWS, _CHUNK_ROWS)], out_sem.at[slot])

        for j in range(min(lookahead, nchunks)):
            in_copy(j, j).start()

        def body(i, carry):
            slot = jax.lax.rem(i, _NSLOTS)
            ahead = i + lookahead
            aslot = jax.lax.rem(ahead, _NSLOTS)

            @pl.when(ahead < nchunks)
            def _():
                @pl.when(ahead - _NSLOTS >= 0)
                def _():
                    out_copy(ahead - _NSLOTS, aslot).wait()
                in_copy(ahead, aslot).start()

            in_copy(i, slot).wait()
            out_copy(i, slot).start()
            return carry

        jax.lax.fori_loop(0, nchunks, body, 0)
        for i in range(max(0, nchunks - _NSLOTS), nchunks):
            out_copy(i, i % _NSLOTS).wait()

    return _copy


def kernel(data, partitions):
    del partitions
    n, d = data.shape
    nchunks = n // _CHUNK_ROWS
    return pl.pallas_call(
        _make_copy_kernel(nchunks),
        in_specs=[pl.BlockSpec(memory_space=pl.ANY)],
        out_specs=pl.BlockSpec(memory_space=pl.ANY),
        out_shape=jax.ShapeDtypeStruct((n, d), data.dtype),
        scratch_shapes=[
            pltpu.VMEM((_NSLOTS, _CHUNK_ROWS, d), jnp.float32),
            pltpu.SemaphoreType.DMA((_NSLOTS,)),
            pltpu.SemaphoreType.DMA((_NSLOTS,)),
        ],
    )(data)
